# Initial kernel scaffold; baseline (speedup 1.0000x reference)
import jax
import jax.numpy as jnp
from jax.experimental import pallas as pl


def kernel(x, edge_index, Wl0, bl0, Wr0, g0, b0, Wl1, bl1, Wr1, g1, b1, Wl2, bl2, Wr2, g2, b2):
    raise NotImplementedError("write your pallas kernel here")


# speedup vs baseline: 7.3083x; 7.3083x over previous
import functools

import jax
import jax.numpy as jnp
from jax import lax
from jax.experimental import pallas as pl
from jax.experimental.pallas import tpu as pltpu
from jax.experimental.pallas import tpu_sc as plsc

N = 10000
D = 128
E = 320000
NC = 2
NS = 16
NW = NC * NS
EPW = E // NW
K = 125
C = EPW // K
STRIPE_A = 632
LAST0 = STRIPE_A * (NS - 1)
STRIPE_B = N - LAST0
CW = 128


def _each_stripe(sid, fn):
    @pl.when(sid < NS - 1)
    def _():
        fn(pl.multiple_of(sid * STRIPE_A, 8), STRIPE_A)

    @pl.when(sid == NS - 1)
    def _():
        fn(LAST0, STRIPE_B)


def _sc_segment_sum(h, src_r, dst_r, zeros_feat):
    mesh = plsc.VectorSubcoreMesh(core_axis_name="c", subcore_axis_name="s")

    @functools.partial(
        pl.kernel,
        out_type=jax.ShapeDtypeStruct((NC, N, D), jnp.float32),
        mesh=mesh,
        scratch_types=[
            pltpu.VMEM((C, K), jnp.int32),
            pltpu.VMEM((C, K), jnp.int32),
            pltpu.VMEM((K, D), jnp.float32),
            pltpu.VMEM_SHARED((N, D), jnp.float32),
            pltpu.SemaphoreType.DMA,
        ])
    def run(h_hbm, src_hbm, dst_hbm, zf_hbm, out_s, src_v, dst_v, rows_v,
            acc_s, sem):
        cid = lax.axis_index("c")
        sid = lax.axis_index("s")
        wid = sid * NC + cid

        def zero_stripe(r0, ln):
            pltpu.sync_copy(zf_hbm.at[pl.ds(r0, ln)], acc_s.at[pl.ds(r0, ln)])

        _each_stripe(sid, zero_stripe)
        pltpu.sync_copy(src_hbm.at[wid], src_v)
        pltpu.sync_copy(dst_hbm.at[wid], dst_v)
        plsc.subcore_barrier()

        @pl.loop(0, C)
        def _(j):
            pltpu.async_copy(h_hbm.at[src_v.at[j]], rows_v, sem).wait()
            pltpu.sync_copy(rows_v, acc_s.at[dst_v.at[j]], add=True)

        plsc.subcore_barrier()

        def readout_stripe(r0, ln):
            pltpu.sync_copy(acc_s.at[pl.ds(r0, ln)],
                            out_s.at[cid, pl.ds(r0, ln)])

        _each_stripe(sid, readout_stripe)

    return run(h, src_r, dst_r, zeros_feat)


def _sc_counts(dst_r, zeros_cnt, ones_cnt):
    mesh = plsc.VectorSubcoreMesh(core_axis_name="c", subcore_axis_name="s")

    @functools.partial(
        pl.kernel,
        out_type=jax.ShapeDtypeStruct((NC, N, CW), jnp.float32),
        mesh=mesh,
        scratch_types=[
            pltpu.VMEM((C, K), jnp.int32),
            pltpu.VMEM((K, CW), jnp.float32),
            pltpu.VMEM_SHARED((N, CW), jnp.float32),
            pltpu.SemaphoreType.DMA,
        ])
    def run(dst_hbm, zc_hbm, ones_hbm, out_c, dst_v, ones_v, acc_c, sem):
        cid = lax.axis_index("c")
        sid = lax.axis_index("s")
        wid = sid * NC + cid

        def zero_stripe(r0, ln):
            pltpu.sync_copy(zc_hbm.at[pl.ds(r0, ln)], acc_c.at[pl.ds(r0, ln)])

        _each_stripe(sid, zero_stripe)
        pltpu.sync_copy(ones_hbm, ones_v)
        pltpu.sync_copy(dst_hbm.at[wid], dst_v)
        plsc.subcore_barrier()

        @pl.loop(0, C)
        def _(j):
            pltpu.sync_copy(ones_v, acc_c.at[dst_v.at[j]], add=True)

        plsc.subcore_barrier()

        def readout_stripe(r0, ln):
            pltpu.sync_copy(acc_c.at[pl.ds(r0, ln)],
                            out_c.at[cid, pl.ds(r0, ln)])

        _each_stripe(sid, readout_stripe)

    return run(dst_r, zeros_cnt, ones_cnt)


def _tc_dense(s_parts, c_parts, x, wl_t, bl, wr_t, g, b, final):
    BLK = 2000
    grid = (N // BLK,)

    def body(s_ref, c_ref, x_ref, wl_ref, bl_ref, wr_ref, g_ref, b_ref,
             o_ref):
        s = s_ref[0] + s_ref[1]
        cnt = c_ref[0, :, 0:1] + c_ref[1, :, 0:1]
        agg = s / jnp.maximum(cnt, 1.0)
        h = (jnp.dot(agg, wl_ref[...], preferred_element_type=jnp.float32)
             + bl_ref[...]
             + jnp.dot(x_ref[...], wr_ref[...],
                       preferred_element_type=jnp.float32))
        mu = jnp.mean(h, axis=1, keepdims=True)
        var = jnp.mean((h - mu) ** 2, axis=1, keepdims=True)
        hn = (h - mu) * lax.rsqrt(var + 1e-5) * g_ref[...] + b_ref[...]
        if final:
            nrm = jnp.sqrt(jnp.sum(hn * hn, axis=1, keepdims=True))
            o_ref[...] = hn / jnp.maximum(nrm, 1e-12)
        else:
            o_ref[...] = jnp.where(hn > 0, hn, jnp.exp(jnp.minimum(hn, 0.0)) - 1.0)

    return pl.pallas_call(
        body,
        grid=grid,
        in_specs=[
            pl.BlockSpec((NC, BLK, D), lambda i: (0, i, 0)),
            pl.BlockSpec((NC, BLK, CW), lambda i: (0, i, 0)),
            pl.BlockSpec((BLK, D), lambda i: (i, 0)),
            pl.BlockSpec((D, D), lambda i: (0, 0)),
            pl.BlockSpec((1, D), lambda i: (0, 0)),
            pl.BlockSpec((D, D), lambda i: (0, 0)),
            pl.BlockSpec((1, D), lambda i: (0, 0)),
            pl.BlockSpec((1, D), lambda i: (0, 0)),
        ],
        out_specs=pl.BlockSpec((BLK, D), lambda i: (i, 0)),
        out_shape=jax.ShapeDtypeStruct((N, D), jnp.float32),
    )(s_parts, c_parts, x, wl_t, bl, wr_t, g, b)


def kernel(x, edge_index, Wl0, bl0, Wr0, g0, b0, Wl1, bl1, Wr1, g1, b1,
           Wl2, bl2, Wr2, g2, b2):
    ei = edge_index.astype(jnp.int32)
    src_r = ei[0].reshape(NW, C, K)
    dst_r = ei[1].reshape(NW, C, K)
    zeros_feat = jnp.zeros((N, D), jnp.float32)
    zeros_cnt = jnp.zeros((N, CW), jnp.float32)
    ones_cnt = jnp.ones((K, CW), jnp.float32)

    c_parts = _sc_counts(dst_r, zeros_cnt, ones_cnt)
    s_parts = _sc_segment_sum(x, src_r, dst_r, zeros_feat)
    h = _tc_dense(s_parts, c_parts, x, Wl0.T, bl0.reshape(1, D), Wr0.T,
                  g0.reshape(1, D), b0.reshape(1, D), final=False)
    s_parts = _sc_segment_sum(h, src_r, dst_r, zeros_feat)
    h = _tc_dense(s_parts, c_parts, h, Wl1.T, bl1.reshape(1, D), Wr1.T,
                  g1.reshape(1, D), b1.reshape(1, D), final=False)
    s_parts = _sc_segment_sum(h, src_r, dst_r, zeros_feat)
    h = _tc_dense(s_parts, c_parts, h, Wl2.T, bl2.reshape(1, D), Wr2.T,
                  g2.reshape(1, D), b2.reshape(1, D), final=True)
    return h


# speedup vs baseline: 10.3454x; 1.4156x over previous
import functools

import jax
import jax.numpy as jnp
from jax import lax
from jax.experimental import pallas as pl
from jax.experimental.pallas import tpu as pltpu
from jax.experimental.pallas import tpu_sc as plsc

N = 10000
D = 128
E = 320000
NC = 2
NS = 16
NW = NC * NS
EPW = E // NW
K = 125
C = EPW // K
STRIPE_A = 632
LAST0 = STRIPE_A * (NS - 1)
STRIPE_B = N - LAST0
CW = 128
NB = 2
NSEC = 2
SECC = C // NSEC


def _each_stripe(sid, fn):
    @pl.when(sid < NS - 1)
    def _():
        fn(pl.multiple_of(sid * STRIPE_A, 8), STRIPE_A)

    @pl.when(sid == NS - 1)
    def _():
        fn(LAST0, STRIPE_B)


def _sc_segment_sum(h, src_r, dst_r, zeros_feat):
    mesh = plsc.VectorSubcoreMesh(core_axis_name="c", subcore_axis_name="s")

    @functools.partial(
        pl.kernel,
        out_type=jax.ShapeDtypeStruct((NC, N, D), jnp.float32),
        mesh=mesh,
        scratch_types=[
            pltpu.VMEM((SECC, K), jnp.int32),
            pltpu.VMEM((SECC, K), jnp.int32),
        ] + [pltpu.VMEM((K, D), jnp.float32) for _ in range(NB)]
          + [pltpu.VMEM_SHARED((N, D), jnp.float32)]
          + [pltpu.SemaphoreType.DMA for _ in range(NB)])
    def run(h_hbm, src_hbm, dst_hbm, zf_hbm, out_s, src_v, dst_v, *rest):
        rows_v = rest[:NB]
        acc_s = rest[NB]
        sems = rest[NB + 1:]
        cid = lax.axis_index("c")
        sid = lax.axis_index("s")
        wid = sid * NC + cid

        def zero_stripe(r0, ln):
            pltpu.sync_copy(zf_hbm.at[pl.ds(r0, ln)], acc_s.at[pl.ds(r0, ln)])

        _each_stripe(sid, zero_stripe)
        plsc.subcore_barrier()

        for sec in range(NSEC):
            pltpu.sync_copy(src_hbm.at[wid, pl.ds(sec * SECC, SECC)], src_v)
            pltpu.sync_copy(dst_hbm.at[wid, pl.ds(sec * SECC, SECC)], dst_v)
            for b in range(NB):
                pltpu.async_copy(h_hbm.at[src_v.at[b]], rows_v[b], sems[b])

            @pl.loop(0, SECC, step=NB)
            def _(j):
                for b in range(NB):
                    pltpu.make_async_copy(
                        h_hbm.at[src_v.at[j + b]], rows_v[b], sems[b]).wait()
                    pltpu.sync_copy(rows_v[b], acc_s.at[dst_v.at[j + b]],
                                    add=True)
                    nxt = j + b + NB

                    @pl.when(nxt < SECC)
                    def _():
                        pltpu.async_copy(h_hbm.at[src_v.at[nxt]], rows_v[b],
                                         sems[b])

        plsc.subcore_barrier()

        def readout_stripe(r0, ln):
            pltpu.sync_copy(acc_s.at[pl.ds(r0, ln)],
                            out_s.at[cid, pl.ds(r0, ln)])

        _each_stripe(sid, readout_stripe)

    return run(h, src_r, dst_r, zeros_feat)


def _sc_counts(dst_r, zeros_cnt, ones_cnt):
    mesh = plsc.VectorSubcoreMesh(core_axis_name="c", subcore_axis_name="s")

    @functools.partial(
        pl.kernel,
        out_type=jax.ShapeDtypeStruct((NC, N, CW), jnp.float32),
        mesh=mesh,
        scratch_types=[
            pltpu.VMEM((C, K), jnp.int32),
            pltpu.VMEM((K, CW), jnp.float32),
            pltpu.VMEM_SHARED((N, CW), jnp.float32),
            pltpu.SemaphoreType.DMA,
        ])
    def run(dst_hbm, zc_hbm, ones_hbm, out_c, dst_v, ones_v, acc_c, sem):
        cid = lax.axis_index("c")
        sid = lax.axis_index("s")
        wid = sid * NC + cid

        def zero_stripe(r0, ln):
            pltpu.sync_copy(zc_hbm.at[pl.ds(r0, ln)], acc_c.at[pl.ds(r0, ln)])

        _each_stripe(sid, zero_stripe)
        pltpu.sync_copy(ones_hbm, ones_v)
        pltpu.sync_copy(dst_hbm.at[wid], dst_v)
        plsc.subcore_barrier()

        @pl.loop(0, C)
        def _(j):
            pltpu.sync_copy(ones_v, acc_c.at[dst_v.at[j]], add=True)

        plsc.subcore_barrier()

        def readout_stripe(r0, ln):
            pltpu.sync_copy(acc_c.at[pl.ds(r0, ln)],
                            out_c.at[cid, pl.ds(r0, ln)])

        _each_stripe(sid, readout_stripe)

    return run(dst_r, zeros_cnt, ones_cnt)


def _tc_dense(s_parts, c_parts, x, wl_t, bl, wr_t, g, b, final):
    BLK = 2000
    grid = (N // BLK,)

    def body(s_ref, c_ref, x_ref, wl_ref, bl_ref, wr_ref, g_ref, b_ref,
             o_ref):
        s = s_ref[0] + s_ref[1]
        cnt = c_ref[0, :, 0:1] + c_ref[1, :, 0:1]
        agg = s / jnp.maximum(cnt, 1.0)
        h = (jnp.dot(agg, wl_ref[...], preferred_element_type=jnp.float32)
             + bl_ref[...]
             + jnp.dot(x_ref[...], wr_ref[...],
                       preferred_element_type=jnp.float32))
        mu = jnp.mean(h, axis=1, keepdims=True)
        var = jnp.mean((h - mu) ** 2, axis=1, keepdims=True)
        hn = (h - mu) * lax.rsqrt(var + 1e-5) * g_ref[...] + b_ref[...]
        if final:
            nrm = jnp.sqrt(jnp.sum(hn * hn, axis=1, keepdims=True))
            o_ref[...] = hn / jnp.maximum(nrm, 1e-12)
        else:
            o_ref[...] = jnp.where(hn > 0, hn, jnp.exp(jnp.minimum(hn, 0.0)) - 1.0)

    return pl.pallas_call(
        body,
        grid=grid,
        in_specs=[
            pl.BlockSpec((NC, BLK, D), lambda i: (0, i, 0)),
            pl.BlockSpec((NC, BLK, CW), lambda i: (0, i, 0)),
            pl.BlockSpec((BLK, D), lambda i: (i, 0)),
            pl.BlockSpec((D, D), lambda i: (0, 0)),
            pl.BlockSpec((1, D), lambda i: (0, 0)),
            pl.BlockSpec((D, D), lambda i: (0, 0)),
            pl.BlockSpec((1, D), lambda i: (0, 0)),
            pl.BlockSpec((1, D), lambda i: (0, 0)),
        ],
        out_specs=pl.BlockSpec((BLK, D), lambda i: (i, 0)),
        out_shape=jax.ShapeDtypeStruct((N, D), jnp.float32),
    )(s_parts, c_parts, x, wl_t, bl, wr_t, g, b)


def kernel(x, edge_index, Wl0, bl0, Wr0, g0, b0, Wl1, bl1, Wr1, g1, b1,
           Wl2, bl2, Wr2, g2, b2):
    ei = edge_index.astype(jnp.int32)
    src_r = ei[0].reshape(NW, C, K)
    dst_r = ei[1].reshape(NW, C, K)
    zeros_feat = jnp.zeros((N, D), jnp.float32)
    zeros_cnt = jnp.zeros((N, CW), jnp.float32)
    ones_cnt = jnp.ones((K, CW), jnp.float32)

    c_parts = _sc_counts(dst_r, zeros_cnt, ones_cnt)
    s_parts = _sc_segment_sum(x, src_r, dst_r, zeros_feat)
    h = _tc_dense(s_parts, c_parts, x, Wl0.T, bl0.reshape(1, D), Wr0.T,
                  g0.reshape(1, D), b0.reshape(1, D), final=False)
    s_parts = _sc_segment_sum(h, src_r, dst_r, zeros_feat)
    h = _tc_dense(s_parts, c_parts, h, Wl1.T, bl1.reshape(1, D), Wr1.T,
                  g1.reshape(1, D), b1.reshape(1, D), final=False)
    s_parts = _sc_segment_sum(h, src_r, dst_r, zeros_feat)
    h = _tc_dense(s_parts, c_parts, h, Wl2.T, bl2.reshape(1, D), Wr2.T,
                  g2.reshape(1, D), b2.reshape(1, D), final=True)
    return h


# speedup vs baseline: 10.3885x; 1.0042x over previous
import functools

import jax
import jax.numpy as jnp
from jax import lax
from jax.experimental import pallas as pl
from jax.experimental.pallas import tpu as pltpu
from jax.experimental.pallas import tpu_sc as plsc

N = 10000
D = 128
E = 320000
NC = 2
NS = 16
NW = NC * NS
EPW = E // NW
K = 50
NB = 5
NSEC = 5
SECC = EPW // (K * NSEC)
CK = 125
CC = EPW // CK
STRIPE_A = 632
LAST0 = STRIPE_A * (NS - 1)
STRIPE_B = N - LAST0
CW = 128


def _each_stripe(sid, fn):
    @pl.when(sid < NS - 1)
    def _():
        fn(pl.multiple_of(sid * STRIPE_A, 8), STRIPE_A)

    @pl.when(sid == NS - 1)
    def _():
        fn(LAST0, STRIPE_B)


def _sc_segment_sum(h, src_r, dst_r, zeros_feat):
    mesh = plsc.VectorSubcoreMesh(core_axis_name="c", subcore_axis_name="s")

    @functools.partial(
        pl.kernel,
        out_type=jax.ShapeDtypeStruct((NC, N, D), jnp.float32),
        mesh=mesh,
        scratch_types=[
            pltpu.VMEM((SECC, K), jnp.int32),
            pltpu.VMEM((SECC, K), jnp.int32),
        ] + [pltpu.VMEM((K, D), jnp.float32) for _ in range(NB)]
          + [pltpu.VMEM_SHARED((N, D), jnp.float32)]
          + [pltpu.SemaphoreType.DMA for _ in range(NB)])
    def run(h_hbm, src_hbm, dst_hbm, zf_hbm, out_s, src_v, dst_v, *rest):
        rows_v = rest[:NB]
        acc_s = rest[NB]
        sems = rest[NB + 1:]
        cid = lax.axis_index("c")
        sid = lax.axis_index("s")
        wid = sid * NC + cid

        def zero_stripe(r0, ln):
            pltpu.sync_copy(zf_hbm.at[pl.ds(r0, ln)], acc_s.at[pl.ds(r0, ln)])

        _each_stripe(sid, zero_stripe)
        plsc.subcore_barrier()

        for sec in range(NSEC):
            pltpu.sync_copy(src_hbm.at[wid, sec], src_v)
            pltpu.sync_copy(dst_hbm.at[wid, sec], dst_v)
            for b in range(NB):
                pltpu.async_copy(h_hbm.at[src_v.at[b]], rows_v[b], sems[b])

            @pl.loop(0, SECC, step=NB)
            def _(j):
                for b in range(NB):
                    pltpu.make_async_copy(
                        h_hbm.at[src_v.at[j + b]], rows_v[b], sems[b]).wait()
                    pltpu.sync_copy(rows_v[b], acc_s.at[dst_v.at[j + b]],
                                    add=True)
                    nxt = j + b + NB

                    @pl.when(nxt < SECC)
                    def _():
                        pltpu.async_copy(h_hbm.at[src_v.at[nxt]], rows_v[b],
                                         sems[b])

        plsc.subcore_barrier()

        def readout_stripe(r0, ln):
            pltpu.sync_copy(acc_s.at[pl.ds(r0, ln)],
                            out_s.at[cid, pl.ds(r0, ln)])

        _each_stripe(sid, readout_stripe)

    return run(h, src_r, dst_r, zeros_feat)


def _sc_counts(dst_r, zeros_cnt, ones_cnt):
    mesh = plsc.VectorSubcoreMesh(core_axis_name="c", subcore_axis_name="s")

    @functools.partial(
        pl.kernel,
        out_type=jax.ShapeDtypeStruct((NC, N, CW), jnp.float32),
        mesh=mesh,
        scratch_types=[
            pltpu.VMEM((CC, CK), jnp.int32),
            pltpu.VMEM((CK, CW), jnp.float32),
            pltpu.VMEM_SHARED((N, CW), jnp.float32),
            pltpu.SemaphoreType.DMA,
        ])
    def run(dst_hbm, zc_hbm, ones_hbm, out_c, dst_v, ones_v, acc_c, sem):
        cid = lax.axis_index("c")
        sid = lax.axis_index("s")
        wid = sid * NC + cid

        def zero_stripe(r0, ln):
            pltpu.sync_copy(zc_hbm.at[pl.ds(r0, ln)], acc_c.at[pl.ds(r0, ln)])

        _each_stripe(sid, zero_stripe)
        pltpu.sync_copy(ones_hbm, ones_v)
        pltpu.sync_copy(dst_hbm.at[wid], dst_v)
        plsc.subcore_barrier()

        @pl.loop(0, CC)
        def _(j):
            pltpu.sync_copy(ones_v, acc_c.at[dst_v.at[j]], add=True)

        plsc.subcore_barrier()

        def readout_stripe(r0, ln):
            pltpu.sync_copy(acc_c.at[pl.ds(r0, ln)],
                            out_c.at[cid, pl.ds(r0, ln)])

        _each_stripe(sid, readout_stripe)

    return run(dst_r, zeros_cnt, ones_cnt)


def _tc_dense(s_parts, c_parts, x, wl_t, bl, wr_t, g, b, final):
    BLK = 2000
    grid = (N // BLK,)

    def body(s_ref, c_ref, x_ref, wl_ref, bl_ref, wr_ref, g_ref, b_ref,
             o_ref):
        s = s_ref[0] + s_ref[1]
        cnt = c_ref[0, :, 0:1] + c_ref[1, :, 0:1]
        agg = s / jnp.maximum(cnt, 1.0)
        h = (jnp.dot(agg, wl_ref[...], preferred_element_type=jnp.float32)
             + bl_ref[...]
             + jnp.dot(x_ref[...], wr_ref[...],
                       preferred_element_type=jnp.float32))
        mu = jnp.mean(h, axis=1, keepdims=True)
        var = jnp.mean((h - mu) ** 2, axis=1, keepdims=True)
        hn = (h - mu) * lax.rsqrt(var + 1e-5) * g_ref[...] + b_ref[...]
        if final:
            nrm = jnp.sqrt(jnp.sum(hn * hn, axis=1, keepdims=True))
            o_ref[...] = hn / jnp.maximum(nrm, 1e-12)
        else:
            o_ref[...] = jnp.where(hn > 0, hn, jnp.exp(jnp.minimum(hn, 0.0)) - 1.0)

    return pl.pallas_call(
        body,
        grid=grid,
        in_specs=[
            pl.BlockSpec((NC, BLK, D), lambda i: (0, i, 0)),
            pl.BlockSpec((NC, BLK, CW), lambda i: (0, i, 0)),
            pl.BlockSpec((BLK, D), lambda i: (i, 0)),
            pl.BlockSpec((D, D), lambda i: (0, 0)),
            pl.BlockSpec((1, D), lambda i: (0, 0)),
            pl.BlockSpec((D, D), lambda i: (0, 0)),
            pl.BlockSpec((1, D), lambda i: (0, 0)),
            pl.BlockSpec((1, D), lambda i: (0, 0)),
        ],
        out_specs=pl.BlockSpec((BLK, D), lambda i: (i, 0)),
        out_shape=jax.ShapeDtypeStruct((N, D), jnp.float32),
    )(s_parts, c_parts, x, wl_t, bl, wr_t, g, b)


def kernel(x, edge_index, Wl0, bl0, Wr0, g0, b0, Wl1, bl1, Wr1, g1, b1,
           Wl2, bl2, Wr2, g2, b2):
    ei = edge_index.astype(jnp.int32)
    src_r = ei[0].reshape(NW, NSEC, SECC, K)
    dst_r = ei[1].reshape(NW, NSEC, SECC, K)
    dst_c = ei[1].reshape(NW, CC, CK)
    zeros_feat = jnp.zeros((N, D), jnp.float32)
    zeros_cnt = jnp.zeros((N, CW), jnp.float32)
    ones_cnt = jnp.ones((CK, CW), jnp.float32)

    c_parts = _sc_counts(dst_c, zeros_cnt, ones_cnt)
    s_parts = _sc_segment_sum(x, src_r, dst_r, zeros_feat)
    h = _tc_dense(s_parts, c_parts, x, Wl0.T, bl0.reshape(1, D), Wr0.T,
                  g0.reshape(1, D), b0.reshape(1, D), final=False)
    s_parts = _sc_segment_sum(h, src_r, dst_r, zeros_feat)
    h = _tc_dense(s_parts, c_parts, h, Wl1.T, bl1.reshape(1, D), Wr1.T,
                  g1.reshape(1, D), b1.reshape(1, D), final=False)
    s_parts = _sc_segment_sum(h, src_r, dst_r, zeros_feat)
    h = _tc_dense(s_parts, c_parts, h, Wl2.T, bl2.reshape(1, D), Wr2.T,
                  g2.reshape(1, D), b2.reshape(1, D), final=True)
    return h


# speedup vs baseline: 10.3940x; 1.0005x over previous
import functools

import jax
import jax.numpy as jnp
from jax import lax
from jax.experimental import pallas as pl
from jax.experimental.pallas import tpu as pltpu
from jax.experimental.pallas import tpu_sc as plsc

N = 10000
D = 128
E = 320000
NC = 2
NS = 16
NW = NC * NS
EPW = E // NW
K = 50
NB = 5
NSEC = 5
SECC = EPW // (K * NSEC)
CK = 125
CC = EPW // CK
STRIPE_A = 632
LAST0 = STRIPE_A * (NS - 1)
STRIPE_B = N - LAST0
CW = 128


def _each_stripe(sid, fn):
    @pl.when(sid < NS - 1)
    def _():
        fn(pl.multiple_of(sid * STRIPE_A, 8), STRIPE_A)

    @pl.when(sid == NS - 1)
    def _():
        fn(LAST0, STRIPE_B)


def _sc_segment_sum(h, src_r, dst_r, zeros_feat, cnt_args=None):
    mesh = plsc.VectorSubcoreMesh(core_axis_name="c", subcore_axis_name="s")
    with_counts = cnt_args is not None

    out_type = [jax.ShapeDtypeStruct((NC, N, D), jnp.float32)]
    scratch = [
        pltpu.VMEM((SECC, K), jnp.int32),
        pltpu.VMEM((SECC, K), jnp.int32),
    ] + [pltpu.VMEM((K, D), jnp.float32) for _ in range(NB)]
    scratch += [pltpu.VMEM_SHARED((N, D), jnp.float32)]
    if with_counts:
        out_type.append(jax.ShapeDtypeStruct((NC, N, CW), jnp.float32))
        scratch += [pltpu.VMEM((K, CW), jnp.float32),
                    pltpu.VMEM_SHARED((N, CW), jnp.float32)]
    scratch += [pltpu.SemaphoreType.DMA for _ in range(NB)]

    @functools.partial(pl.kernel, out_type=out_type, mesh=mesh,
                       scratch_types=scratch)
    def run(h_hbm, src_hbm, dst_hbm, zf_hbm, *rest):
        if with_counts:
            zc_hbm, ones_hbm = rest[0], rest[1]
            rest = rest[2:]
        out_s = rest[0]
        if with_counts:
            out_c = rest[1]
            rest = rest[2:]
        else:
            rest = rest[1:]
        src_v, dst_v = rest[0], rest[1]
        rows_v = rest[2:2 + NB]
        acc_s = rest[2 + NB]
        rest = rest[3 + NB:]
        if with_counts:
            ones_v, acc_c = rest[0], rest[1]
            rest = rest[2:]
        sems = rest
        cid = lax.axis_index("c")
        sid = lax.axis_index("s")
        wid = sid * NC + cid

        def zero_stripe(r0, ln):
            pltpu.sync_copy(zf_hbm.at[pl.ds(r0, ln)], acc_s.at[pl.ds(r0, ln)])
            if with_counts:
                pltpu.sync_copy(zc_hbm.at[pl.ds(r0, ln)],
                                acc_c.at[pl.ds(r0, ln)])

        _each_stripe(sid, zero_stripe)
        if with_counts:
            pltpu.sync_copy(ones_hbm, ones_v)
        plsc.subcore_barrier()

        for sec in range(NSEC):
            pltpu.sync_copy(src_hbm.at[wid, sec], src_v)
            pltpu.sync_copy(dst_hbm.at[wid, sec], dst_v)
            for b in range(NB):
                pltpu.async_copy(h_hbm.at[src_v.at[b]], rows_v[b], sems[b])

            @pl.loop(0, SECC, step=NB)
            def _(j):
                for b in range(NB):
                    pltpu.make_async_copy(
                        h_hbm.at[src_v.at[j + b]], rows_v[b], sems[b]).wait()
                    pltpu.sync_copy(rows_v[b], acc_s.at[dst_v.at[j + b]],
                                    add=True)
                    if with_counts:
                        pltpu.sync_copy(ones_v, acc_c.at[dst_v.at[j + b]],
                                        add=True)
                    nxt = j + b + NB

                    @pl.when(nxt < SECC)
                    def _():
                        pltpu.async_copy(h_hbm.at[src_v.at[nxt]], rows_v[b],
                                         sems[b])

        plsc.subcore_barrier()

        def readout_stripe(r0, ln):
            pltpu.sync_copy(acc_s.at[pl.ds(r0, ln)],
                            out_s.at[cid, pl.ds(r0, ln)])
            if with_counts:
                pltpu.sync_copy(acc_c.at[pl.ds(r0, ln)],
                                out_c.at[cid, pl.ds(r0, ln)])

        _each_stripe(sid, readout_stripe)

    if with_counts:
        return run(h, src_r, dst_r, zeros_feat, cnt_args[0], cnt_args[1])
    return run(h, src_r, dst_r, zeros_feat)[0]


def _sc_counts(dst_c, zeros_cnt, ones_cnt):
    mesh = plsc.VectorSubcoreMesh(core_axis_name="c", subcore_axis_name="s")

    @functools.partial(
        pl.kernel,
        out_type=[jax.ShapeDtypeStruct((NC, N, CW), jnp.float32)],
        mesh=mesh,
        scratch_types=[
            pltpu.VMEM((CC, CK), jnp.int32),
            pltpu.VMEM((CK, CW), jnp.float32),
            pltpu.VMEM_SHARED((N, CW), jnp.float32),
        ])
    def run(dst_hbm, zc_hbm, ones_hbm, out_c, dst_v, ones_v, acc_c):
        cid = lax.axis_index("c")
        sid = lax.axis_index("s")
        wid = sid * NC + cid

        def zero_stripe(r0, ln):
            pltpu.sync_copy(zc_hbm.at[pl.ds(r0, ln)], acc_c.at[pl.ds(r0, ln)])

        _each_stripe(sid, zero_stripe)
        pltpu.sync_copy(ones_hbm, ones_v)
        pltpu.sync_copy(dst_hbm.at[wid], dst_v)
        plsc.subcore_barrier()

        @pl.loop(0, CC)
        def _(j):
            pltpu.sync_copy(ones_v, acc_c.at[dst_v.at[j]], add=True)

        plsc.subcore_barrier()

        def readout_stripe(r0, ln):
            pltpu.sync_copy(acc_c.at[pl.ds(r0, ln)],
                            out_c.at[cid, pl.ds(r0, ln)])

        _each_stripe(sid, readout_stripe)

    return run(dst_c, zeros_cnt, ones_cnt)[0]


def _tc_dense(s_parts, c_parts, x, wl_t, bl, wr_t, g, b, final):
    BLK = 2000
    grid = (N // BLK,)

    def body(s_ref, c_ref, x_ref, wl_ref, bl_ref, wr_ref, g_ref, b_ref,
             o_ref):
        s = s_ref[0] + s_ref[1]
        cnt = c_ref[0, :, 0:1] + c_ref[1, :, 0:1]
        agg = s / jnp.maximum(cnt, 1.0)
        h = (jnp.dot(agg, wl_ref[...], preferred_element_type=jnp.float32)
             + bl_ref[...]
             + jnp.dot(x_ref[...], wr_ref[...],
                       preferred_element_type=jnp.float32))
        mu = jnp.mean(h, axis=1, keepdims=True)
        var = jnp.mean((h - mu) ** 2, axis=1, keepdims=True)
        hn = (h - mu) * lax.rsqrt(var + 1e-5) * g_ref[...] + b_ref[...]
        if final:
            nrm = jnp.sqrt(jnp.sum(hn * hn, axis=1, keepdims=True))
            o_ref[...] = hn / jnp.maximum(nrm, 1e-12)
        else:
            o_ref[...] = jnp.where(hn > 0, hn, jnp.exp(jnp.minimum(hn, 0.0)) - 1.0)

    return pl.pallas_call(
        body,
        grid=grid,
        in_specs=[
            pl.BlockSpec((NC, BLK, D), lambda i: (0, i, 0)),
            pl.BlockSpec((NC, BLK, CW), lambda i: (0, i, 0)),
            pl.BlockSpec((BLK, D), lambda i: (i, 0)),
            pl.BlockSpec((D, D), lambda i: (0, 0)),
            pl.BlockSpec((1, D), lambda i: (0, 0)),
            pl.BlockSpec((D, D), lambda i: (0, 0)),
            pl.BlockSpec((1, D), lambda i: (0, 0)),
            pl.BlockSpec((1, D), lambda i: (0, 0)),
        ],
        out_specs=pl.BlockSpec((BLK, D), lambda i: (i, 0)),
        out_shape=jax.ShapeDtypeStruct((N, D), jnp.float32),
    )(s_parts, c_parts, x, wl_t, bl, wr_t, g, b)


def kernel(x, edge_index, Wl0, bl0, Wr0, g0, b0, Wl1, bl1, Wr1, g1, b1,
           Wl2, bl2, Wr2, g2, b2):
    ei = edge_index.astype(jnp.int32)
    src_r = ei[0].reshape(NW, NSEC, SECC, K)
    dst_r = ei[1].reshape(NW, NSEC, SECC, K)
    dst_c = ei[1].reshape(NW, CC, CK)
    zeros_feat = jnp.zeros((N, D), jnp.float32)
    zeros_cnt = jnp.zeros((N, CW), jnp.float32)
    ones_cnt = jnp.ones((CK, CW), jnp.float32)

    c_parts = _sc_counts(dst_c, zeros_cnt, ones_cnt)
    s_parts = _sc_segment_sum(x, src_r, dst_r, zeros_feat)
    h = _tc_dense(s_parts, c_parts, x, Wl0.T, bl0.reshape(1, D), Wr0.T,
                  g0.reshape(1, D), b0.reshape(1, D), final=False)
    s_parts = _sc_segment_sum(h, src_r, dst_r, zeros_feat)
    h = _tc_dense(s_parts, c_parts, h, Wl1.T, bl1.reshape(1, D), Wr1.T,
                  g1.reshape(1, D), b1.reshape(1, D), final=False)
    s_parts = _sc_segment_sum(h, src_r, dst_r, zeros_feat)
    h = _tc_dense(s_parts, c_parts, h, Wl2.T, bl2.reshape(1, D), Wr2.T,
                  g2.reshape(1, D), b2.reshape(1, D), final=True)
    return h
